# Initial kernel scaffold; baseline (speedup 1.0000x reference)
import jax
import jax.numpy as jnp
from jax.experimental import pallas as pl


def kernel(position_ids, weights):
    raise NotImplementedError("write your pallas kernel here")


# speedup vs baseline: 2.3916x; 2.3916x over previous
import functools

import jax
import jax.numpy as jnp
from jax import lax
from jax.experimental import pallas as pl
from jax.experimental.pallas import tpu as pltpu
from jax.experimental.pallas import tpu_sc as plsc

NUM_EMB = 8192
EMB_DIM = 1024

NC = 2
NS = 16
NW = NC * NS

B_TOTAL = 4 * 8192
R = B_TOTAL // NW
CHUNK = 32
NBUF = 2
NCHUNK = R // CHUNK
K_OUTER = NCHUNK // NBUF


def _emb_body(idx_hbm, table_hbm, out_hbm, idx_v, buf_v, gsem, ssem):
    wid = lax.axis_index("s") * NC + lax.axis_index("c")
    base = wid * R

    pltpu.sync_copy(idx_hbm.at[pl.ds(base, R)], idx_v)

    def gather_start(i, b):
        pltpu.async_copy(
            table_hbm.at[idx_v.at[pl.ds(i * CHUNK, CHUNK)]],
            buf_v.at[b],
            gsem.at[b],
        )

    def gather_wait(b):
        pltpu.make_async_copy(
            table_hbm.at[idx_v.at[pl.ds(0, CHUNK)]], buf_v.at[b], gsem.at[b]
        ).wait()

    def store_start(i, b):
        pltpu.async_copy(
            buf_v.at[b], out_hbm.at[pl.ds(base + i * CHUNK, CHUNK)], ssem.at[b]
        )

    def store_wait(b):
        pltpu.make_async_copy(
            buf_v.at[b], out_hbm.at[pl.ds(base, CHUNK)], ssem.at[b]
        ).wait()

    for b in range(NBUF):
        gather_start(b, b)

    def outer(k, carry):
        for b in range(NBUF):
            i = k * NBUF + b
            gather_wait(b)
            store_start(i, b)

            @pl.when(k < K_OUTER - 1)
            def _():
                store_wait(b)
                gather_start(i + NBUF, b)

        return carry

    lax.fori_loop(0, K_OUTER, outer, 0)

    for b in range(NBUF):
        store_wait(b)


@functools.partial(jax.jit, static_argnames=())
def _lookup(idx_flat, weights):
    mesh = plsc.VectorSubcoreMesh(core_axis_name="c", subcore_axis_name="s")
    return pl.kernel(
        _emb_body,
        out_type=jax.ShapeDtypeStruct((B_TOTAL, EMB_DIM), jnp.float32),
        mesh=mesh,
        scratch_types=[
            pltpu.VMEM((R,), jnp.int32),
            pltpu.VMEM((NBUF, CHUNK, EMB_DIM), jnp.float32),
            pltpu.SemaphoreType.DMA((NBUF,)),
            pltpu.SemaphoreType.DMA((NBUF,)),
        ],
    )(idx_flat, weights)


def kernel(position_ids, weights):
    batch, length = position_ids.shape
    out = _lookup(position_ids.reshape(-1), weights)
    return out.reshape(batch, length, EMB_DIM)
